# baseline (speedup 1.0000x reference)
import functools

import jax
import jax.numpy as jnp
from jax import lax
from jax.experimental import pallas as pl
from jax.experimental.pallas import tpu as pltpu
from jax.experimental.pallas import tpu_sc as plsc

_B = 16384
_IDX_CHUNK = 128


def _make_sc_neigh(n_workers):
    bw = _B // n_workers
    mesh = plsc.VectorSubcoreMesh(core_axis_name="c", subcore_axis_name="s")

    @functools.partial(
        pl.kernel,
        mesh=mesh,
        compiler_params=pltpu.CompilerParams(
            use_tc_tiling_on_sc=False, needs_layout_passes=False),
        out_type=jax.ShapeDtypeStruct((32, _B), jnp.float32),
        scratch_types=[
            pltpu.VMEM((bw,), jnp.int32),
            pltpu.VMEM((1000, 32), jnp.float32),
            pltpu.VMEM((32, bw), jnp.float32),
            pltpu.SemaphoreType.DMA,
        ],
    )
    def neigh_k(ntab, ids, nt_out, niv, ntab_v, nrowT, tsem):
        wid = lax.axis_index("s") * 2 + lax.axis_index("c")
        base = wid * bw
        tab = pltpu.async_copy(ntab, ntab_v, tsem)
        pltpu.sync_copy(ids.at[pl.ds(_B + base, bw)], niv)
        tab.wait()

        def group_body(g, carry):
            rows = g * 16
            nid = niv[pl.ds(rows, 16)]
            for c in range(32):
                cvec = jnp.full((16,), c, jnp.int32)
                v = plsc.load_gather(ntab_v, [nid, cvec])
                nrowT.at[c][pl.ds(rows, 16)] = v
            return carry

        lax.fori_loop(0, bw // 16, group_body, 0)
        pltpu.sync_copy(nrowT, nt_out.at[:, pl.ds(base, bw)])

    return neigh_k


def _make_sc_city(n_workers):
    bw = _B // n_workers
    nchunk = bw // _IDX_CHUNK
    mesh = plsc.VectorSubcoreMesh(core_axis_name="c", subcore_axis_name="s")

    @functools.partial(
        pl.kernel,
        mesh=mesh,
        compiler_params=pltpu.CompilerParams(
            use_tc_tiling_on_sc=True, needs_layout_passes=False),
        out_type=jax.ShapeDtypeStruct((_B, 128), jnp.float32),
        scratch_types=[
            pltpu.VMEM((bw,), jnp.int32),
            pltpu.VMEM((bw, 128), jnp.float32),
            pltpu.SemaphoreType.DMA,
        ],
    )
    def city_k(ctab, ids, city_out, civ, crow, sem):
        wid = lax.axis_index("s") * 2 + lax.axis_index("c")
        base = wid * bw
        pltpu.sync_copy(ids.at[pl.ds(base, bw)], civ)
        copies = []
        for j in range(nchunk):
            rows = pl.ds(j * _IDX_CHUNK, _IDX_CHUNK)
            copies.append(pltpu.async_copy(
                ctab.at[civ.at[rows]], crow.at[rows], sem))
        for c in copies:
            c.wait()
        pltpu.sync_copy(crow, city_out.at[pl.ds(base, bw)])

    return city_k


def _mlp_body(city_ref, nt_ref, xt_ref, cpar_ref,
              w1c_ref, w1n_ref, w1x_ref, ptw_ref, ttw_ref,
              b1_ref, g1_ref, be1_ref,
              w2_ref, b2_ref, g2_ref, be2_ref, out_ref):
    f32 = jnp.float32
    rb = city_ref.shape[0]
    mask = cpar_ref[...] != 0.0
    city64 = jnp.where(mask, city_ref[:, 64:128], city_ref[:, 0:64])
    h = jnp.dot(city64, w1c_ref[...], preferred_element_type=f32)
    cdims = (((0,), (0,)), ((), ()))
    h = h + lax.dot_general(nt_ref[...], w1n_ref[...],
                            dimension_numbers=cdims,
                            preferred_element_type=f32)
    h = h + lax.dot_general(xt_ref[...], w1x_ref[...],
                            dimension_numbers=cdims,
                            preferred_element_type=f32)
    pid = xt_ref[108:109, :].astype(jnp.int32)
    poh = (lax.broadcasted_iota(jnp.int32, (8, rb), 0) == pid).astype(f32)
    h = h + lax.dot_general(poh, ptw_ref[...], dimension_numbers=cdims,
                            preferred_element_type=f32)
    tid = xt_ref[109:110, :].astype(jnp.int32)
    toh = (lax.broadcasted_iota(jnp.int32, (48, rb), 0) == tid).astype(f32)
    h = h + lax.dot_general(toh, ttw_ref[...], dimension_numbers=cdims,
                            preferred_element_type=f32)
    h = h + b1_ref[...]
    mu = jnp.mean(h, axis=-1, keepdims=True)
    var = jnp.mean((h - mu) * (h - mu), axis=-1, keepdims=True)
    h = (h - mu) * lax.rsqrt(var + 1e-5) * g1_ref[...] + be1_ref[...]
    h = jnp.maximum(h, 0.0)
    o = jnp.dot(h, w2_ref[...], preferred_element_type=f32)
    o = o + b2_ref[...]
    mu2 = jnp.mean(o, axis=-1, keepdims=True)
    var2 = jnp.mean((o - mu2) * (o - mu2), axis=-1, keepdims=True)
    out_ref[...] = (o - mu2) * lax.rsqrt(var2 + 1e-5) * g2_ref[...] + be2_ref[...]


def kernel(x, city_table, neigh_table, price_table, time_table,
           W1, b1, g1, be1, W2, b2, g2, be2):
    f32 = jnp.float32
    city_ids = x[:, 0].astype(jnp.int32)
    ids = jnp.concatenate([
        city_ids >> 1,
        x[:, 1].astype(jnp.int32),
    ])
    cpar = (city_ids & 1).astype(f32).reshape(_B, 1)

    cityv = city_table.reshape(50000, 128)

    info = plsc.get_sparse_core_info()
    n_workers = info.num_cores * info.num_subcores

    neighT = _make_sc_neigh(n_workers)(neigh_table, ids)
    cityp = _make_sc_city(n_workers)(cityv, ids)

    w1c = W1[0:64]
    w1n = W1[64:96]
    z = lambda n: jnp.zeros((n, W1.shape[1]), f32)
    w1x = jnp.concatenate(
        [z(2), W1[120:226], z(2), W1[226:230]], axis=0)
    ptw = price_table @ W1[96:104]
    ttw = time_table @ W1[104:120]

    rb = 1024
    grid = (_B // rb,)
    full = lambda i: (0, 0)
    row = lambda i: (i, 0)
    col = lambda i: (0, i)
    out = pl.pallas_call(
        _mlp_body,
        grid=grid,
        in_specs=[
            pl.BlockSpec((rb, 128), row),
            pl.BlockSpec((32, rb), col),
            pl.BlockSpec((114, rb), col),
            pl.BlockSpec((rb, 1), row),
            pl.BlockSpec((64, 256), full),
            pl.BlockSpec((32, 256), full),
            pl.BlockSpec((114, 256), full),
            pl.BlockSpec((8, 256), full),
            pl.BlockSpec((48, 256), full),
            pl.BlockSpec((1, 256), full),
            pl.BlockSpec((1, 256), full),
            pl.BlockSpec((1, 256), full),
            pl.BlockSpec((256, 128), full),
            pl.BlockSpec((1, 128), full),
            pl.BlockSpec((1, 128), full),
            pl.BlockSpec((1, 128), full),
        ],
        out_specs=pl.BlockSpec((rb, 128), row),
        out_shape=jax.ShapeDtypeStruct((_B, 128), f32),
        compiler_params=pltpu.CompilerParams(
            dimension_semantics=("arbitrary",)),
    )(cityp, neighT, x.T, cpar,
      w1c, w1n, w1x, ptw, ttw,
      b1.reshape(1, -1), g1.reshape(1, -1), be1.reshape(1, -1),
      W2, b2.reshape(1, -1), g2.reshape(1, -1), be2.reshape(1, -1))
    return out


# speedup vs baseline: 1.1139x; 1.1139x over previous
import functools

import jax
import jax.numpy as jnp
from jax import lax
from jax.experimental import pallas as pl
from jax.experimental.pallas import tpu as pltpu
from jax.experimental.pallas import tpu_sc as plsc

_B = 16384
_IDX_CHUNK = 128


def _make_sc_small(n_workers):
    bw = _B // n_workers
    mesh = plsc.VectorSubcoreMesh(core_axis_name="c", subcore_axis_name="s")

    @functools.partial(
        pl.kernel,
        mesh=mesh,
        compiler_params=pltpu.CompilerParams(
            use_tc_tiling_on_sc=False, needs_layout_passes=False),
        out_type=jax.ShapeDtypeStruct((_B, 128), jnp.float32),
        scratch_types=[
            pltpu.VMEM((bw,), jnp.int32),
            pltpu.VMEM((bw,), jnp.int32),
            pltpu.VMEM((bw,), jnp.int32),
            pltpu.VMEM((1000, 32), jnp.float32),
            pltpu.VMEM((8, 16), jnp.float32),
            pltpu.VMEM((48, 16), jnp.float32),
            pltpu.VMEM((bw, 32), jnp.float32),
            pltpu.VMEM((bw, 16), jnp.float32),
            pltpu.VMEM((bw, 16), jnp.float32),
            pltpu.SemaphoreType.DMA,
            pltpu.SemaphoreType.DMA,
        ],
    )
    def small_k(ntab, ptab, ttab, ids, rest_out,
                niv, piv, tiv, ntab_v, ptab_v, ttab_v,
                nrow, prow, trow, tsem, wsem):
        wid = lax.axis_index("s") * 2 + lax.axis_index("c")
        base = wid * bw
        tabs = [
            pltpu.async_copy(ntab, ntab_v, tsem),
            pltpu.async_copy(ptab, ptab_v, tsem),
            pltpu.async_copy(ttab, ttab_v, tsem),
        ]
        pltpu.sync_copy(ids.at[pl.ds(_B + base, bw)], niv)
        pltpu.sync_copy(ids.at[pl.ds(2 * _B + base, bw)], piv)
        pltpu.sync_copy(ids.at[pl.ds(3 * _B + base, bw)], tiv)
        for t in tabs:
            t.wait()

        def group_body(g, carry):
            rows = g * 16
            nid = niv[pl.ds(rows, 16)]
            pid = piv[pl.ds(rows, 16)]
            tid = tiv[pl.ds(rows, 16)]
            rpos = rows + lax.iota(jnp.int32, 16)
            for c in range(32):
                cvec = jnp.full((16,), c, jnp.int32)
                plsc.store_scatter(nrow, [rpos, cvec],
                                   plsc.load_gather(ntab_v, [nid, cvec]))
            for c in range(16):
                cvec = jnp.full((16,), c, jnp.int32)
                plsc.store_scatter(prow, [rpos, cvec],
                                   plsc.load_gather(ptab_v, [pid, cvec]))
                plsc.store_scatter(trow, [rpos, cvec],
                                   plsc.load_gather(ttab_v, [tid, cvec]))
            return carry

        lax.fori_loop(0, bw // 16, group_body, 0)

        out_rows = pl.ds(base, bw)
        writes = [
            pltpu.async_copy(nrow, rest_out.at[out_rows, pl.ds(0, 32)], wsem),
            pltpu.async_copy(prow, rest_out.at[out_rows, pl.ds(32, 16)], wsem),
            pltpu.async_copy(trow, rest_out.at[out_rows, pl.ds(48, 16)], wsem),
            pltpu.async_copy(nrow, rest_out.at[out_rows, pl.ds(64, 32)], wsem),
            pltpu.async_copy(prow, rest_out.at[out_rows, pl.ds(96, 16)], wsem),
            pltpu.async_copy(trow, rest_out.at[out_rows, pl.ds(112, 16)], wsem),
        ]
        for w in writes:
            w.wait()

    return small_k


def _make_sc_city(n_workers):
    bw = _B // n_workers
    nchunk = bw // _IDX_CHUNK
    mesh = plsc.VectorSubcoreMesh(core_axis_name="c", subcore_axis_name="s")

    @functools.partial(
        pl.kernel,
        mesh=mesh,
        compiler_params=pltpu.CompilerParams(
            use_tc_tiling_on_sc=False, needs_layout_passes=False),
        out_type=jax.ShapeDtypeStruct((_B, 128), jnp.float32),
        scratch_types=[
            pltpu.VMEM((bw,), jnp.int32),
            pltpu.VMEM((bw, 64), jnp.float32),
            pltpu.SemaphoreType.DMA,
            pltpu.SemaphoreType.DMA,
        ],
    )
    def city_k(ctab, ids, city_out, civ, crow, sem, wsem):
        wid = lax.axis_index("s") * 2 + lax.axis_index("c")
        base = wid * bw
        pltpu.sync_copy(ids.at[pl.ds(base, bw)], civ)
        copies = []
        for j in range(nchunk):
            rows = pl.ds(j * _IDX_CHUNK, _IDX_CHUNK)
            copies.append(pltpu.async_copy(
                ctab.at[civ.at[rows]], crow.at[rows], sem))
        for c in copies:
            c.wait()
        out_rows = pl.ds(base, bw)
        writes = [
            pltpu.async_copy(crow, city_out.at[out_rows, pl.ds(0, 64)], wsem),
            pltpu.async_copy(crow, city_out.at[out_rows, pl.ds(64, 64)], wsem),
        ]
        for w in writes:
            w.wait()

    return city_k


def _mlp_body(city_ref, rest_ref, x_ref,
              w1c_ref, w1r_ref, w1x_ref, b1_ref, g1_ref, be1_ref,
              w2_ref, b2_ref, g2_ref, be2_ref, out_ref):
    h = jnp.dot(city_ref[...], w1c_ref[...], preferred_element_type=jnp.float32)
    h = h + jnp.dot(rest_ref[...], w1r_ref[...],
                    preferred_element_type=jnp.float32)
    h = h + jnp.dot(x_ref[...], w1x_ref[...],
                    preferred_element_type=jnp.float32)
    h = h + b1_ref[...]
    mu = jnp.mean(h, axis=-1, keepdims=True)
    var = jnp.mean((h - mu) * (h - mu), axis=-1, keepdims=True)
    h = (h - mu) * lax.rsqrt(var + 1e-5) * g1_ref[...] + be1_ref[...]
    h = jnp.maximum(h, 0.0)
    o = jnp.dot(h, w2_ref[...], preferred_element_type=jnp.float32)
    o = o + b2_ref[...]
    mu2 = jnp.mean(o, axis=-1, keepdims=True)
    var2 = jnp.mean((o - mu2) * (o - mu2), axis=-1, keepdims=True)
    out_ref[...] = (o - mu2) * lax.rsqrt(var2 + 1e-5) * g2_ref[...] + be2_ref[...]


def kernel(x, city_table, neigh_table, price_table, time_table,
           W1, b1, g1, be1, W2, b2, g2, be2):
    f32 = jnp.float32
    ids = jnp.concatenate([
        x[:, 0].astype(jnp.int32),
        x[:, 1].astype(jnp.int32),
        x[:, 108].astype(jnp.int32),
        x[:, 109].astype(jnp.int32),
    ])

    price_pad = jnp.pad(price_table, ((0, 0), (0, 8)))

    info = plsc.get_sparse_core_info()
    n_workers = info.num_cores * info.num_subcores

    rest = _make_sc_small(n_workers)(neigh_table, price_pad, time_table, ids)
    cityp = _make_sc_city(n_workers)(city_table, ids)

    z = lambda n: jnp.zeros((n, W1.shape[1]), f32)
    w1c = jnp.concatenate([W1[0:64], z(64)], axis=0)
    w1r = jnp.concatenate(
        [W1[64:96], W1[96:104], z(8), W1[104:120], z(64)], axis=0)
    w1x = jnp.concatenate(
        [z(2), W1[120:226], z(2), W1[226:230]], axis=0)

    rb = 2048
    grid = (_B // rb,)
    full = lambda i: (0, 0)
    row = lambda i: (i, 0)
    out = pl.pallas_call(
        _mlp_body,
        grid=grid,
        in_specs=[
            pl.BlockSpec((rb, 128), row),
            pl.BlockSpec((rb, 128), row),
            pl.BlockSpec((rb, 114), row),
            pl.BlockSpec((128, 256), full),
            pl.BlockSpec((128, 256), full),
            pl.BlockSpec((114, 256), full),
            pl.BlockSpec((1, 256), full),
            pl.BlockSpec((1, 256), full),
            pl.BlockSpec((1, 256), full),
            pl.BlockSpec((256, 128), full),
            pl.BlockSpec((1, 128), full),
            pl.BlockSpec((1, 128), full),
            pl.BlockSpec((1, 128), full),
        ],
        out_specs=pl.BlockSpec((rb, 128), row),
        out_shape=jax.ShapeDtypeStruct((_B, 128), f32),
        compiler_params=pltpu.CompilerParams(
            dimension_semantics=("arbitrary",)),
    )(cityp, rest, x,
      w1c, w1r, w1x, b1.reshape(1, -1), g1.reshape(1, -1), be1.reshape(1, -1),
      W2, b2.reshape(1, -1), g2.reshape(1, -1), be2.reshape(1, -1))
    return out


# speedup vs baseline: 1.1306x; 1.0150x over previous
import functools

import jax
import jax.numpy as jnp
from jax import lax
from jax.experimental import pallas as pl
from jax.experimental.pallas import tpu as pltpu
from jax.experimental.pallas import tpu_sc as plsc

_B = 16384
_IDX_CHUNK = 128


def _make_sc_small(n_workers):
    bw = _B // n_workers
    mesh = plsc.VectorSubcoreMesh(core_axis_name="c", subcore_axis_name="s")

    @functools.partial(
        pl.kernel,
        mesh=mesh,
        compiler_params=pltpu.CompilerParams(
            use_tc_tiling_on_sc=False, needs_layout_passes=False),
        out_type=jax.ShapeDtypeStruct((_B, 128), jnp.float32),
        scratch_types=[
            pltpu.VMEM((bw,), jnp.int32),
            pltpu.VMEM((bw,), jnp.int32),
            pltpu.VMEM((bw,), jnp.int32),
            pltpu.VMEM((1000, 32), jnp.float32),
            pltpu.VMEM((8, 16), jnp.float32),
            pltpu.VMEM((48, 16), jnp.float32),
            pltpu.VMEM((bw, 32), jnp.float32),
            pltpu.VMEM((bw, 16), jnp.float32),
            pltpu.VMEM((bw, 16), jnp.float32),
            pltpu.SemaphoreType.DMA,
            pltpu.SemaphoreType.DMA,
        ],
    )
    def small_k(ntab, ptab, ttab, ids, rest_out,
                niv, piv, tiv, ntab_v, ptab_v, ttab_v,
                nrow, prow, trow, tsem, wsem):
        wid = lax.axis_index("s") * 2 + lax.axis_index("c")
        base = wid * bw
        tabs = [
            pltpu.async_copy(ntab, ntab_v, tsem),
            pltpu.async_copy(ptab, ptab_v, tsem),
            pltpu.async_copy(ttab, ttab_v, tsem),
        ]
        pltpu.sync_copy(ids.at[pl.ds(_B + base, bw)], niv)
        pltpu.sync_copy(ids.at[pl.ds(2 * _B + base, bw)], piv)
        pltpu.sync_copy(ids.at[pl.ds(3 * _B + base, bw)], tiv)
        for t in tabs:
            t.wait()

        def group_body(g, carry):
            rows = g * 16
            nid = niv[pl.ds(rows, 16)]
            pid = piv[pl.ds(rows, 16)]
            tid = tiv[pl.ds(rows, 16)]
            rpos = rows + lax.iota(jnp.int32, 16)
            for c in range(32):
                cvec = jnp.full((16,), c, jnp.int32)
                plsc.store_scatter(nrow, [rpos, cvec],
                                   plsc.load_gather(ntab_v, [nid, cvec]))
            for c in range(16):
                cvec = jnp.full((16,), c, jnp.int32)
                plsc.store_scatter(prow, [rpos, cvec],
                                   plsc.load_gather(ptab_v, [pid, cvec]))
                plsc.store_scatter(trow, [rpos, cvec],
                                   plsc.load_gather(ttab_v, [tid, cvec]))
            return carry

        lax.fori_loop(0, bw // 16, group_body, 0)

        out_rows = pl.ds(base, bw)
        writes = [
            pltpu.async_copy(nrow, rest_out.at[out_rows, pl.ds(0, 32)], wsem),
            pltpu.async_copy(prow, rest_out.at[out_rows, pl.ds(32, 16)], wsem),
            pltpu.async_copy(trow, rest_out.at[out_rows, pl.ds(48, 16)], wsem),
        ]
        for w in writes:
            w.wait()

    return small_k


def _make_sc_city(n_workers):
    bw = _B // n_workers
    nchunk = bw // _IDX_CHUNK
    mesh = plsc.VectorSubcoreMesh(core_axis_name="c", subcore_axis_name="s")

    @functools.partial(
        pl.kernel,
        mesh=mesh,
        compiler_params=pltpu.CompilerParams(
            use_tc_tiling_on_sc=False, needs_layout_passes=False),
        out_type=jax.ShapeDtypeStruct((_B, 128), jnp.float32),
        scratch_types=[
            pltpu.VMEM((bw,), jnp.int32),
            pltpu.VMEM((bw, 64), jnp.float32),
            pltpu.SemaphoreType.DMA,
            pltpu.SemaphoreType.DMA,
        ],
    )
    def city_k(ctab, ids, city_out, civ, crow, sem, wsem):
        wid = lax.axis_index("s") * 2 + lax.axis_index("c")
        base = wid * bw
        pltpu.sync_copy(ids.at[pl.ds(base, bw)], civ)
        copies = []
        for j in range(nchunk):
            rows = pl.ds(j * _IDX_CHUNK, _IDX_CHUNK)
            copies.append(pltpu.async_copy(
                ctab.at[civ.at[rows]], crow.at[rows], sem))
        for c in copies:
            c.wait()
        pltpu.async_copy(
            crow, city_out.at[pl.ds(base, bw), pl.ds(0, 64)], wsem).wait()

    return city_k


def _mlp_body(city_ref, rest_ref, x_ref,
              w1c_ref, w1r_ref, w1x_ref, b1_ref, g1_ref, be1_ref,
              w2_ref, b2_ref, g2_ref, be2_ref, out_ref):
    h = jnp.dot(city_ref[:, 0:64], w1c_ref[...],
                preferred_element_type=jnp.float32)
    h = h + jnp.dot(rest_ref[:, 0:64], w1r_ref[...],
                    preferred_element_type=jnp.float32)
    h = h + jnp.dot(x_ref[...], w1x_ref[...],
                    preferred_element_type=jnp.float32)
    h = h + b1_ref[...]
    mu = jnp.mean(h, axis=-1, keepdims=True)
    var = jnp.mean((h - mu) * (h - mu), axis=-1, keepdims=True)
    h = (h - mu) * lax.rsqrt(var + 1e-5) * g1_ref[...] + be1_ref[...]
    h = jnp.maximum(h, 0.0)
    o = jnp.dot(h, w2_ref[...], preferred_element_type=jnp.float32)
    o = o + b2_ref[...]
    mu2 = jnp.mean(o, axis=-1, keepdims=True)
    var2 = jnp.mean((o - mu2) * (o - mu2), axis=-1, keepdims=True)
    out_ref[...] = (o - mu2) * lax.rsqrt(var2 + 1e-5) * g2_ref[...] + be2_ref[...]


def kernel(x, city_table, neigh_table, price_table, time_table,
           W1, b1, g1, be1, W2, b2, g2, be2):
    f32 = jnp.float32
    ids = jnp.concatenate([
        x[:, 0].astype(jnp.int32),
        x[:, 1].astype(jnp.int32),
        x[:, 108].astype(jnp.int32),
        x[:, 109].astype(jnp.int32),
    ])

    price_pad = jnp.pad(price_table, ((0, 0), (0, 8)))

    info = plsc.get_sparse_core_info()
    n_workers = info.num_cores * info.num_subcores

    rest = _make_sc_small(n_workers)(neigh_table, price_pad, time_table, ids)
    cityp = _make_sc_city(n_workers)(city_table, ids)

    z = lambda n: jnp.zeros((n, W1.shape[1]), f32)
    w1c = W1[0:64]
    w1r = jnp.concatenate(
        [W1[64:96], W1[96:104], z(8), W1[104:120]], axis=0)
    w1x = jnp.concatenate(
        [z(2), W1[120:226], z(2), W1[226:230]], axis=0)

    rb = 2048
    grid = (_B // rb,)
    full = lambda i: (0, 0)
    row = lambda i: (i, 0)
    out = pl.pallas_call(
        _mlp_body,
        grid=grid,
        in_specs=[
            pl.BlockSpec((rb, 128), row),
            pl.BlockSpec((rb, 128), row),
            pl.BlockSpec((rb, 114), row),
            pl.BlockSpec((64, 256), full),
            pl.BlockSpec((64, 256), full),
            pl.BlockSpec((114, 256), full),
            pl.BlockSpec((1, 256), full),
            pl.BlockSpec((1, 256), full),
            pl.BlockSpec((1, 256), full),
            pl.BlockSpec((256, 128), full),
            pl.BlockSpec((1, 128), full),
            pl.BlockSpec((1, 128), full),
            pl.BlockSpec((1, 128), full),
        ],
        out_specs=pl.BlockSpec((rb, 128), row),
        out_shape=jax.ShapeDtypeStruct((_B, 128), f32),
        compiler_params=pltpu.CompilerParams(
            dimension_semantics=("arbitrary",)),
    )(cityp, rest, x,
      w1c, w1r, w1x, b1.reshape(1, -1), g1.reshape(1, -1), be1.reshape(1, -1),
      W2, b2.reshape(1, -1), g2.reshape(1, -1), be2.reshape(1, -1))
    return out
